# Initial kernel scaffold; baseline (speedup 1.0000x reference)
import functools

import jax
import jax.numpy as jnp
from jax import lax
from jax.experimental import pallas as pl
from jax.experimental.pallas import tpu as pltpu
from jax.experimental.pallas import tpu_sc as plsc

N_NODES = 10000
N_EDGES = 320000
D_IN = 128
D_HID = 128
N_CLASSES = 64

NC = 2
NS = 16
CH = 80
CHUNKS_PER_TILE = N_EDGES // (NC * NS * CH)
ROWS_PER_TILE = N_NODES // NS
DEG_W = 16
BR = 2000
GRID = N_NODES // BR

_mesh = plsc.VectorSubcoreMesh(
    core_axis_name="c", subcore_axis_name="s", num_cores=NC, num_subcores=NS
)


def _deg_body(src_hbm, dst_hbm, z16_hbm, ones_hbm, out_hbm,
              sidx, didx, ones_v, s_tbl, d_tbl):
    c = lax.axis_index("c")
    s = lax.axis_index("s")
    rbase = s * ROWS_PER_TILE
    rows = pl.ds(rbase, ROWS_PER_TILE)
    pltpu.sync_copy(z16_hbm.at[rows], s_tbl.at[rows])
    pltpu.sync_copy(z16_hbm.at[rows], d_tbl.at[rows])
    pltpu.sync_copy(ones_hbm, ones_v)
    cbase = (c * NS + s) * CHUNKS_PER_TILE
    pltpu.sync_copy(src_hbm.at[pl.ds(cbase, CHUNKS_PER_TILE)], sidx)
    pltpu.sync_copy(dst_hbm.at[pl.ds(cbase, CHUNKS_PER_TILE)], didx)
    plsc.subcore_barrier()

    @pl.loop(0, CHUNKS_PER_TILE)
    def _(j):
        pltpu.sync_copy(ones_v, s_tbl.at[sidx.at[j]], add=True)
        pltpu.sync_copy(ones_v, d_tbl.at[didx.at[j]], add=True)

    plsc.subcore_barrier()
    pltpu.sync_copy(s_tbl.at[rows], out_hbm.at[c, 0, rows])
    pltpu.sync_copy(d_tbl.at[rows], out_hbm.at[c, 1, rows])


_deg_kernel = pl.kernel(
    _deg_body,
    out_type=jax.ShapeDtypeStruct((NC, 2, N_NODES, DEG_W), jnp.float32),
    mesh=_mesh,
    scratch_types=[
        pltpu.VMEM((CHUNKS_PER_TILE, CH), jnp.int32),
        pltpu.VMEM((CHUNKS_PER_TILE, CH), jnp.int32),
        pltpu.VMEM((CH, DEG_W), jnp.float32),
        pltpu.VMEM_SHARED((N_NODES, DEG_W), jnp.float32),
        pltpu.VMEM_SHARED((N_NODES, DEG_W), jnp.float32),
    ],
)


def _gs_body(h_hbm, src_hbm, dst_hbm, z_hbm, out_hbm, sidx, didx, rows_v, acc):
    c = lax.axis_index("c")
    s = lax.axis_index("s")
    rbase = s * ROWS_PER_TILE
    rows = pl.ds(rbase, ROWS_PER_TILE)
    pltpu.sync_copy(z_hbm.at[rows], acc.at[rows])
    cbase = (c * NS + s) * CHUNKS_PER_TILE
    pltpu.sync_copy(src_hbm.at[pl.ds(cbase, CHUNKS_PER_TILE)], sidx)
    pltpu.sync_copy(dst_hbm.at[pl.ds(cbase, CHUNKS_PER_TILE)], didx)
    plsc.subcore_barrier()

    @pl.loop(0, CHUNKS_PER_TILE)
    def _(j):
        pltpu.sync_copy(h_hbm.at[sidx.at[j]], rows_v)
        pltpu.sync_copy(rows_v, acc.at[didx.at[j]], add=True)

    plsc.subcore_barrier()
    pltpu.sync_copy(acc.at[rows], out_hbm.at[c, rows])


def _make_gs(width):
    return pl.kernel(
        _gs_body,
        out_type=jax.ShapeDtypeStruct((NC, N_NODES, width), jnp.float32),
        mesh=_mesh,
        scratch_types=[
            pltpu.VMEM((CHUNKS_PER_TILE, CH), jnp.int32),
            pltpu.VMEM((CHUNKS_PER_TILE, CH), jnp.int32),
            pltpu.VMEM((CH, width), jnp.float32),
            pltpu.VMEM_SHARED((N_NODES, width), jnp.float32),
        ],
    )


_gs128 = _make_gs(D_IN)
_gs64 = _make_gs(N_CLASSES)


def _inv_sqrt_deg(degs_ref, kind):
    d = degs_ref[0, kind][:, :1] + degs_ref[1, kind][:, :1]
    return lax.rsqrt(jnp.maximum(d, 1.0))


def _prescale_body(x_ref, degs_ref, h_ref):
    h_ref[...] = x_ref[...] * _inv_sqrt_deg(degs_ref, 0)


def _mid_body(p_ref, degs_ref, w1_ref, b1_ref, w2_ref, g_ref):
    agg = (p_ref[0] + p_ref[1]) * _inv_sqrt_deg(degs_ref, 1)
    t = jnp.dot(agg, w1_ref[...], preferred_element_type=jnp.float32)
    t = jnp.maximum(t + b1_ref[...], 0.0) * _inv_sqrt_deg(degs_ref, 0)
    g_ref[...] = jnp.dot(t, w2_ref[...], preferred_element_type=jnp.float32)


def _final_body(q_ref, degs_ref, b2_ref, o_ref):
    o_ref[...] = (q_ref[0] + q_ref[1]) * _inv_sqrt_deg(degs_ref, 1) + b2_ref[...]


_DEG_SPEC = pl.BlockSpec((NC, 2, BR, DEG_W), lambda i: (0, 0, i, 0))


def _prescale(x, degs):
    return pl.pallas_call(
        _prescale_body,
        out_shape=jax.ShapeDtypeStruct((N_NODES, D_IN), jnp.float32),
        grid=(GRID,),
        in_specs=[pl.BlockSpec((BR, D_IN), lambda i: (i, 0)), _DEG_SPEC],
        out_specs=pl.BlockSpec((BR, D_IN), lambda i: (i, 0)),
    )(x, degs)


def _mid(p, degs, W1, b1, W2):
    return pl.pallas_call(
        _mid_body,
        out_shape=jax.ShapeDtypeStruct((N_NODES, N_CLASSES), jnp.float32),
        grid=(GRID,),
        in_specs=[
            pl.BlockSpec((NC, BR, D_IN), lambda i: (0, i, 0)),
            _DEG_SPEC,
            pl.BlockSpec((D_IN, D_HID), lambda i: (0, 0)),
            pl.BlockSpec((1, D_HID), lambda i: (0, 0)),
            pl.BlockSpec((D_HID, N_CLASSES), lambda i: (0, 0)),
        ],
        out_specs=pl.BlockSpec((BR, N_CLASSES), lambda i: (i, 0)),
    )(p, degs, W1, b1, W2)


def _final(q, degs, b2):
    return pl.pallas_call(
        _final_body,
        out_shape=jax.ShapeDtypeStruct((N_NODES, N_CLASSES), jnp.float32),
        grid=(GRID,),
        in_specs=[
            pl.BlockSpec((NC, BR, N_CLASSES), lambda i: (0, i, 0)),
            _DEG_SPEC,
            pl.BlockSpec((1, N_CLASSES), lambda i: (0, 0)),
        ],
        out_specs=pl.BlockSpec((BR, N_CLASSES), lambda i: (i, 0)),
    )(q, degs, b2)


def kernel(x, edge_index, W1, b1, W2, b2):
    ei = edge_index.astype(jnp.int32)
    src2d = ei[0].reshape(N_EDGES // CH, CH)
    dst2d = ei[1].reshape(N_EDGES // CH, CH)
    z16 = jnp.zeros((N_NODES, DEG_W), jnp.float32)
    z128 = jnp.zeros((N_NODES, D_IN), jnp.float32)
    z64 = jnp.zeros((N_NODES, N_CLASSES), jnp.float32)
    ones16 = jnp.ones((CH, DEG_W), jnp.float32)

    degs = _deg_kernel(src2d, dst2d, z16, ones16)
    h1 = _prescale(x, degs)
    p1 = _gs128(h1, src2d, dst2d, z128)
    g = _mid(p1, degs, W1, b1.reshape(1, D_HID), W2)
    p2 = _gs64(g, src2d, dst2d, z64)
    return _final(p2, degs, b2.reshape(1, N_CLASSES))


# speedup vs baseline: 8.1256x; 8.1256x over previous
import functools

import jax
import jax.numpy as jnp
from jax import lax
from jax.experimental import pallas as pl
from jax.experimental.pallas import tpu as pltpu
from jax.experimental.pallas import tpu_sc as plsc

N_NODES = 10000
N_EDGES = 320000
D_IN = 128
D_HID = 128
N_CLASSES = 64

NC = 2
NS = 16
CH = 80
CHUNKS_PER_TILE = N_EDGES // (NC * NS * CH)
ROWS_PER_TILE = 10240 // NS
DEG_W = 16
N_PAD = 10240
BR = 2000
GRID = N_NODES // BR

_mesh = plsc.VectorSubcoreMesh(
    core_axis_name="c", subcore_axis_name="s", num_cores=NC, num_subcores=NS
)


def _deg_body(src_hbm, dst_hbm, z16_hbm, ones_hbm, out_hbm,
              sidx, didx, ones_v, s_tbl, d_tbl):
    c = lax.axis_index("c")
    s = lax.axis_index("s")
    rbase = s * ROWS_PER_TILE
    rows = pl.ds(rbase, ROWS_PER_TILE)
    pltpu.sync_copy(z16_hbm.at[rows], s_tbl.at[rows])
    pltpu.sync_copy(z16_hbm.at[rows], d_tbl.at[rows])
    pltpu.sync_copy(ones_hbm, ones_v)
    wid = c * NS + s
    pltpu.sync_copy(src_hbm.at[wid], sidx)
    pltpu.sync_copy(dst_hbm.at[wid], didx)
    plsc.subcore_barrier()

    @pl.loop(0, CHUNKS_PER_TILE)
    def _(j):
        pltpu.sync_copy(ones_v, s_tbl.at[sidx.at[j]], add=True)
        pltpu.sync_copy(ones_v, d_tbl.at[didx.at[j]], add=True)

    plsc.subcore_barrier()
    pltpu.sync_copy(s_tbl.at[rows], out_hbm.at[c, 0, rows])
    pltpu.sync_copy(d_tbl.at[rows], out_hbm.at[c, 1, rows])


_deg_kernel = pl.kernel(
    _deg_body,
    out_type=jax.ShapeDtypeStruct((NC, 2, N_PAD, DEG_W), jnp.float32),
    mesh=_mesh,
    compiler_params=pltpu.CompilerParams(use_tc_tiling_on_sc=False),
    scratch_types=[
        pltpu.VMEM((CHUNKS_PER_TILE, CH), jnp.int32),
        pltpu.VMEM((CHUNKS_PER_TILE, CH), jnp.int32),
        pltpu.VMEM((CH, DEG_W), jnp.float32),
        pltpu.VMEM_SHARED((N_PAD, DEG_W), jnp.float32),
        pltpu.VMEM_SHARED((N_PAD, DEG_W), jnp.float32),
    ],
)


def _gs_body(h_hbm, src_hbm, dst_hbm, z_hbm, out_hbm, sidx, didx, rows_v, acc):
    c = lax.axis_index("c")
    s = lax.axis_index("s")
    rbase = s * ROWS_PER_TILE
    rows = pl.ds(rbase, ROWS_PER_TILE)
    pltpu.sync_copy(z_hbm.at[rows], acc.at[rows])
    wid = c * NS + s
    pltpu.sync_copy(src_hbm.at[wid], sidx)
    pltpu.sync_copy(dst_hbm.at[wid], didx)
    plsc.subcore_barrier()

    @pl.loop(0, CHUNKS_PER_TILE)
    def _(j):
        pltpu.sync_copy(h_hbm.at[sidx.at[j]], rows_v)
        pltpu.sync_copy(rows_v, acc.at[didx.at[j]], add=True)

    plsc.subcore_barrier()
    pltpu.sync_copy(acc.at[rows], out_hbm.at[c, rows])


def _make_gs(width):
    return pl.kernel(
        _gs_body,
        out_type=jax.ShapeDtypeStruct((NC, N_PAD, width), jnp.float32),
        mesh=_mesh,
        compiler_params=pltpu.CompilerParams(use_tc_tiling_on_sc=False),
        scratch_types=[
            pltpu.VMEM((CHUNKS_PER_TILE, CH), jnp.int32),
            pltpu.VMEM((CHUNKS_PER_TILE, CH), jnp.int32),
            pltpu.VMEM((CH, width), jnp.float32),
            pltpu.VMEM_SHARED((N_PAD, width), jnp.float32),
        ],
    )


_gs128 = _make_gs(D_IN)
_gs64 = _make_gs(N_CLASSES)


def _inv_sqrt_deg(degs_ref, kind):
    d = degs_ref[0, kind][:, :1] + degs_ref[1, kind][:, :1]
    return lax.rsqrt(jnp.maximum(d, 1.0))


def _prescale_body(x_ref, degs_ref, h_ref):
    h_ref[...] = x_ref[...] * _inv_sqrt_deg(degs_ref, 0)


def _mid_body(p_ref, degs_ref, w1_ref, b1_ref, w2_ref, g_ref):
    agg = (p_ref[0] + p_ref[1]) * _inv_sqrt_deg(degs_ref, 1)
    t = jnp.dot(agg, w1_ref[...], preferred_element_type=jnp.float32)
    t = jnp.maximum(t + b1_ref[...], 0.0) * _inv_sqrt_deg(degs_ref, 0)
    g_ref[...] = jnp.dot(t, w2_ref[...], preferred_element_type=jnp.float32)


def _final_body(q_ref, degs_ref, b2_ref, o_ref):
    o_ref[...] = (q_ref[0] + q_ref[1]) * _inv_sqrt_deg(degs_ref, 1) + b2_ref[...]


_DEG_SPEC = pl.BlockSpec((NC, 2, BR, DEG_W), lambda i: (0, 0, i, 0))


def _prescale(x, degs):
    return pl.pallas_call(
        _prescale_body,
        out_shape=jax.ShapeDtypeStruct((N_NODES, D_IN), jnp.float32),
        grid=(GRID,),
        in_specs=[pl.BlockSpec((BR, D_IN), lambda i: (i, 0)), _DEG_SPEC],
        out_specs=pl.BlockSpec((BR, D_IN), lambda i: (i, 0)),
    )(x, degs)


def _mid(p, degs, W1, b1, W2):
    return pl.pallas_call(
        _mid_body,
        out_shape=jax.ShapeDtypeStruct((N_NODES, N_CLASSES), jnp.float32),
        grid=(GRID,),
        in_specs=[
            pl.BlockSpec((NC, BR, D_IN), lambda i: (0, i, 0)),
            _DEG_SPEC,
            pl.BlockSpec((D_IN, D_HID), lambda i: (0, 0)),
            pl.BlockSpec((1, D_HID), lambda i: (0, 0)),
            pl.BlockSpec((D_HID, N_CLASSES), lambda i: (0, 0)),
        ],
        out_specs=pl.BlockSpec((BR, N_CLASSES), lambda i: (i, 0)),
    )(p, degs, W1, b1, W2)


def _final(q, degs, b2):
    return pl.pallas_call(
        _final_body,
        out_shape=jax.ShapeDtypeStruct((N_NODES, N_CLASSES), jnp.float32),
        grid=(GRID,),
        in_specs=[
            pl.BlockSpec((NC, BR, N_CLASSES), lambda i: (0, i, 0)),
            _DEG_SPEC,
            pl.BlockSpec((1, N_CLASSES), lambda i: (0, 0)),
        ],
        out_specs=pl.BlockSpec((BR, N_CLASSES), lambda i: (i, 0)),
    )(q, degs, b2)


def kernel(x, edge_index, W1, b1, W2, b2):
    ei = edge_index.astype(jnp.int32)
    src2d = ei[0].reshape(NC * NS, CHUNKS_PER_TILE, CH)
    dst2d = ei[1].reshape(NC * NS, CHUNKS_PER_TILE, CH)
    z16 = jnp.zeros((N_PAD, DEG_W), jnp.float32)
    z128 = jnp.zeros((N_PAD, D_IN), jnp.float32)
    z64 = jnp.zeros((N_PAD, N_CLASSES), jnp.float32)
    ones16 = jnp.ones((CH, DEG_W), jnp.float32)

    degs = _deg_kernel(src2d, dst2d, z16, ones16)
    h1 = _prescale(x, degs)
    p1 = _gs128(h1, src2d, dst2d, z128)
    g = _mid(p1, degs, W1, b1.reshape(1, D_HID), W2)
    p2 = _gs64(g, src2d, dst2d, z64)
    return _final(p2, degs, b2.reshape(1, N_CLASSES))


# speedup vs baseline: 11.2131x; 1.3800x over previous
import functools

import jax
import jax.numpy as jnp
from jax import lax
from jax.experimental import pallas as pl
from jax.experimental.pallas import tpu as pltpu
from jax.experimental.pallas import tpu_sc as plsc

N_NODES = 10000
N_EDGES = 320000
D_IN = 128
D_HID = 128
N_CLASSES = 64

NC = 2
NS = 16
CH = 80
CHUNKS_PER_TILE = N_EDGES // (NC * NS * CH)
ROWS_PER_TILE = 10240 // NS
DEG_W = 16
N_PAD = 10240
BR = 2000
GRID = N_NODES // BR

_mesh = plsc.VectorSubcoreMesh(
    core_axis_name="c", subcore_axis_name="s", num_cores=NC, num_subcores=NS
)


def _deg_body(src_hbm, dst_hbm, z16_hbm, ones_hbm, out_hbm,
              sidx, didx, ones_v, s_tbl, d_tbl):
    c = lax.axis_index("c")
    s = lax.axis_index("s")
    rbase = s * ROWS_PER_TILE
    rows = pl.ds(rbase, ROWS_PER_TILE)
    pltpu.sync_copy(z16_hbm.at[rows], s_tbl.at[rows])
    pltpu.sync_copy(z16_hbm.at[rows], d_tbl.at[rows])
    pltpu.sync_copy(ones_hbm, ones_v)
    wid = c * NS + s
    pltpu.sync_copy(src_hbm.at[wid], sidx)
    pltpu.sync_copy(dst_hbm.at[wid], didx)
    plsc.subcore_barrier()

    @pl.loop(0, CHUNKS_PER_TILE)
    def _(j):
        pltpu.sync_copy(ones_v, s_tbl.at[sidx.at[j]], add=True)
        pltpu.sync_copy(ones_v, d_tbl.at[didx.at[j]], add=True)

    plsc.subcore_barrier()
    pltpu.sync_copy(s_tbl.at[rows], out_hbm.at[c, 0, rows])
    pltpu.sync_copy(d_tbl.at[rows], out_hbm.at[c, 1, rows])


_deg_kernel = pl.kernel(
    _deg_body,
    out_type=jax.ShapeDtypeStruct((NC, 2, N_PAD, DEG_W), jnp.float32),
    mesh=_mesh,
    compiler_params=pltpu.CompilerParams(use_tc_tiling_on_sc=False),
    scratch_types=[
        pltpu.VMEM((CHUNKS_PER_TILE, CH), jnp.int32),
        pltpu.VMEM((CHUNKS_PER_TILE, CH), jnp.int32),
        pltpu.VMEM((CH, DEG_W), jnp.float32),
        pltpu.VMEM_SHARED((N_PAD, DEG_W), jnp.float32),
        pltpu.VMEM_SHARED((N_PAD, DEG_W), jnp.float32),
    ],
)


def _gs_body(nb, h_hbm, src_hbm, dst_hbm, z_hbm, out_hbm, sidx, didx, rows_v,
             gsem, ssem, acc):
    c = lax.axis_index("c")
    s = lax.axis_index("s")
    rbase = s * ROWS_PER_TILE
    rows = pl.ds(rbase, ROWS_PER_TILE)
    pltpu.sync_copy(z_hbm.at[rows], acc.at[rows])
    wid = c * NS + s
    pltpu.sync_copy(src_hbm.at[wid], sidx)
    pltpu.sync_copy(dst_hbm.at[wid], didx)
    plsc.subcore_barrier()

    npipe = CHUNKS_PER_TILE // nb

    for b in range(nb):
        pltpu.async_copy(h_hbm.at[sidx.at[b]], rows_v.at[b], gsem.at[b])

    @pl.loop(0, npipe - 1)
    def _(t):
        j0 = t * nb
        scat = []
        for b in range(nb):
            pltpu.make_async_copy(
                h_hbm.at[sidx.at[j0 + b]], rows_v.at[b], gsem.at[b]).wait()
            scat.append(pltpu.async_copy(
                rows_v.at[b], acc.at[didx.at[j0 + b]], ssem.at[b], add=True))
        for b in range(nb):
            scat[b].wait()
            pltpu.async_copy(
                h_hbm.at[sidx.at[j0 + nb + b]], rows_v.at[b], gsem.at[b])

    j0 = (npipe - 1) * nb
    scat = []
    for b in range(nb):
        pltpu.make_async_copy(
            h_hbm.at[sidx.at[j0 + b]], rows_v.at[b], gsem.at[b]).wait()
        scat.append(pltpu.async_copy(
            rows_v.at[b], acc.at[didx.at[j0 + b]], ssem.at[b], add=True))
    for d in scat:
        d.wait()
    for j in range(npipe * nb, CHUNKS_PER_TILE):
        pltpu.sync_copy(h_hbm.at[sidx.at[j]], rows_v.at[0])
        pltpu.sync_copy(rows_v.at[0], acc.at[didx.at[j]], add=True)

    plsc.subcore_barrier()
    pltpu.sync_copy(acc.at[rows], out_hbm.at[c, rows])


def _make_gs(width, nb):
    return pl.kernel(
        functools.partial(_gs_body, nb),
        out_type=jax.ShapeDtypeStruct((NC, N_PAD, width), jnp.float32),
        mesh=_mesh,
        compiler_params=pltpu.CompilerParams(use_tc_tiling_on_sc=False),
        scratch_types=[
            pltpu.VMEM((CHUNKS_PER_TILE, CH), jnp.int32),
            pltpu.VMEM((CHUNKS_PER_TILE, CH), jnp.int32),
            pltpu.VMEM((nb, CH, width), jnp.float32),
            pltpu.SemaphoreType.DMA((nb,)),
            pltpu.SemaphoreType.DMA((nb,)),
            pltpu.VMEM_SHARED((N_PAD, width), jnp.float32),
        ],
    )


_gs128 = _make_gs(D_IN, 2)
_gs64 = _make_gs(N_CLASSES, 5)


def _inv_sqrt_deg(degs_ref, kind):
    d = degs_ref[0, kind][:, :1] + degs_ref[1, kind][:, :1]
    return lax.rsqrt(jnp.maximum(d, 1.0))


def _prescale_body(x_ref, degs_ref, h_ref):
    h_ref[...] = x_ref[...] * _inv_sqrt_deg(degs_ref, 0)


def _mid_body(p_ref, degs_ref, w1_ref, b1_ref, w2_ref, g_ref):
    agg = (p_ref[0] + p_ref[1]) * _inv_sqrt_deg(degs_ref, 1)
    t = jnp.dot(agg, w1_ref[...], preferred_element_type=jnp.float32)
    t = jnp.maximum(t + b1_ref[...], 0.0) * _inv_sqrt_deg(degs_ref, 0)
    g_ref[...] = jnp.dot(t, w2_ref[...], preferred_element_type=jnp.float32)


def _final_body(q_ref, degs_ref, b2_ref, o_ref):
    o_ref[...] = (q_ref[0] + q_ref[1]) * _inv_sqrt_deg(degs_ref, 1) + b2_ref[...]


_DEG_SPEC = pl.BlockSpec((NC, 2, BR, DEG_W), lambda i: (0, 0, i, 0))


def _prescale(x, degs):
    return pl.pallas_call(
        _prescale_body,
        out_shape=jax.ShapeDtypeStruct((N_NODES, D_IN), jnp.float32),
        grid=(GRID,),
        in_specs=[pl.BlockSpec((BR, D_IN), lambda i: (i, 0)), _DEG_SPEC],
        out_specs=pl.BlockSpec((BR, D_IN), lambda i: (i, 0)),
    )(x, degs)


def _mid(p, degs, W1, b1, W2):
    return pl.pallas_call(
        _mid_body,
        out_shape=jax.ShapeDtypeStruct((N_NODES, N_CLASSES), jnp.float32),
        grid=(GRID,),
        in_specs=[
            pl.BlockSpec((NC, BR, D_IN), lambda i: (0, i, 0)),
            _DEG_SPEC,
            pl.BlockSpec((D_IN, D_HID), lambda i: (0, 0)),
            pl.BlockSpec((1, D_HID), lambda i: (0, 0)),
            pl.BlockSpec((D_HID, N_CLASSES), lambda i: (0, 0)),
        ],
        out_specs=pl.BlockSpec((BR, N_CLASSES), lambda i: (i, 0)),
    )(p, degs, W1, b1, W2)


def _final(q, degs, b2):
    return pl.pallas_call(
        _final_body,
        out_shape=jax.ShapeDtypeStruct((N_NODES, N_CLASSES), jnp.float32),
        grid=(GRID,),
        in_specs=[
            pl.BlockSpec((NC, BR, N_CLASSES), lambda i: (0, i, 0)),
            _DEG_SPEC,
            pl.BlockSpec((1, N_CLASSES), lambda i: (0, 0)),
        ],
        out_specs=pl.BlockSpec((BR, N_CLASSES), lambda i: (i, 0)),
    )(q, degs, b2)


def kernel(x, edge_index, W1, b1, W2, b2):
    ei = edge_index.astype(jnp.int32)
    src2d = ei[0].reshape(NC * NS, CHUNKS_PER_TILE, CH)
    dst2d = ei[1].reshape(NC * NS, CHUNKS_PER_TILE, CH)
    z16 = jnp.zeros((N_PAD, DEG_W), jnp.float32)
    z128 = jnp.zeros((N_PAD, D_IN), jnp.float32)
    z64 = jnp.zeros((N_PAD, N_CLASSES), jnp.float32)
    ones16 = jnp.ones((CH, DEG_W), jnp.float32)

    degs = _deg_kernel(src2d, dst2d, z16, ones16)
    h1 = _prescale(x, degs)
    p1 = _gs128(h1, src2d, dst2d, z128)
    g = _mid(p1, degs, W1, b1.reshape(1, D_HID), W2)
    p2 = _gs64(g, src2d, dst2d, z64)
    return _final(p2, degs, b2.reshape(1, N_CLASSES))


# speedup vs baseline: 12.1653x; 1.0849x over previous
import functools

import jax
import jax.numpy as jnp
from jax import lax
from jax.experimental import pallas as pl
from jax.experimental.pallas import tpu as pltpu
from jax.experimental.pallas import tpu_sc as plsc

N_NODES = 10000
N_EDGES = 320000
D_IN = 128
D_HID = 128
N_CLASSES = 64

NC = 2
NS = 16
CH = 80
CHUNKS_PER_TILE = N_EDGES // (NC * NS * CH)
ROWS_PER_TILE = 10240 // NS
DEG_W = 16
N_PAD = 10240
BR = 2000
GRID = N_NODES // BR

_mesh = plsc.VectorSubcoreMesh(
    core_axis_name="c", subcore_axis_name="s", num_cores=NC, num_subcores=NS
)


DEG_NB = 5


def _deg_body(src_hbm, dst_hbm, z16_hbm, ones_hbm, out_hbm,
              sidx, didx, ones_v, ssem, dsem, s_tbl, d_tbl):
    c = lax.axis_index("c")
    s = lax.axis_index("s")
    rbase = s * ROWS_PER_TILE
    rows = pl.ds(rbase, ROWS_PER_TILE)
    pltpu.sync_copy(z16_hbm.at[rows], s_tbl.at[rows])
    pltpu.sync_copy(z16_hbm.at[rows], d_tbl.at[rows])
    pltpu.sync_copy(ones_hbm, ones_v)
    wid = c * NS + s
    pltpu.sync_copy(src_hbm.at[wid], sidx)
    pltpu.sync_copy(dst_hbm.at[wid], didx)
    plsc.subcore_barrier()

    for b in range(DEG_NB):
        pltpu.async_copy(ones_v, s_tbl.at[sidx.at[b]], ssem.at[b], add=True)
        pltpu.async_copy(ones_v, d_tbl.at[didx.at[b]], dsem.at[b], add=True)

    @pl.loop(1, CHUNKS_PER_TILE // DEG_NB)
    def _(t):
        j0 = t * DEG_NB
        for b in range(DEG_NB):
            pltpu.make_async_copy(
                ones_v, s_tbl.at[sidx.at[j0 - DEG_NB + b]], ssem.at[b]).wait()
            pltpu.make_async_copy(
                ones_v, d_tbl.at[didx.at[j0 - DEG_NB + b]], dsem.at[b]).wait()
            pltpu.async_copy(ones_v, s_tbl.at[sidx.at[j0 + b]], ssem.at[b],
                             add=True)
            pltpu.async_copy(ones_v, d_tbl.at[didx.at[j0 + b]], dsem.at[b],
                             add=True)

    j0 = CHUNKS_PER_TILE - DEG_NB
    for b in range(DEG_NB):
        pltpu.make_async_copy(ones_v, s_tbl.at[sidx.at[j0 + b]],
                              ssem.at[b]).wait()
        pltpu.make_async_copy(ones_v, d_tbl.at[didx.at[j0 + b]],
                              dsem.at[b]).wait()

    plsc.subcore_barrier()
    pltpu.sync_copy(s_tbl.at[rows], out_hbm.at[c, 0, rows])
    pltpu.sync_copy(d_tbl.at[rows], out_hbm.at[c, 1, rows])


_deg_kernel = pl.kernel(
    _deg_body,
    out_type=jax.ShapeDtypeStruct((NC, 2, N_PAD, DEG_W), jnp.float32),
    mesh=_mesh,
    compiler_params=pltpu.CompilerParams(use_tc_tiling_on_sc=False),
    scratch_types=[
        pltpu.VMEM((CHUNKS_PER_TILE, CH), jnp.int32),
        pltpu.VMEM((CHUNKS_PER_TILE, CH), jnp.int32),
        pltpu.VMEM((CH, DEG_W), jnp.float32),
        pltpu.SemaphoreType.DMA((DEG_NB,)),
        pltpu.SemaphoreType.DMA((DEG_NB,)),
        pltpu.VMEM_SHARED((N_PAD, DEG_W), jnp.float32),
        pltpu.VMEM_SHARED((N_PAD, DEG_W), jnp.float32),
    ],
)


def _gs_body(nb, h_hbm, src_hbm, dst_hbm, z_hbm, out_hbm, sidx, didx, rows_v,
             gsem, ssem, acc):
    c = lax.axis_index("c")
    s = lax.axis_index("s")
    rbase = s * ROWS_PER_TILE
    rows = pl.ds(rbase, ROWS_PER_TILE)
    pltpu.sync_copy(z_hbm.at[rows], acc.at[rows])
    wid = c * NS + s
    pltpu.sync_copy(src_hbm.at[wid], sidx)
    pltpu.sync_copy(dst_hbm.at[wid], didx)
    plsc.subcore_barrier()

    npipe = CHUNKS_PER_TILE // nb

    for b in range(nb):
        pltpu.async_copy(h_hbm.at[sidx.at[b]], rows_v.at[b], gsem.at[b])

    @pl.loop(0, npipe - 1)
    def _(t):
        j0 = t * nb
        scat = []
        for b in range(nb):
            pltpu.make_async_copy(
                h_hbm.at[sidx.at[j0 + b]], rows_v.at[b], gsem.at[b]).wait()
            scat.append(pltpu.async_copy(
                rows_v.at[b], acc.at[didx.at[j0 + b]], ssem.at[b], add=True))
        for b in range(nb):
            scat[b].wait()
            pltpu.async_copy(
                h_hbm.at[sidx.at[j0 + nb + b]], rows_v.at[b], gsem.at[b])

    j0 = (npipe - 1) * nb
    scat = []
    for b in range(nb):
        pltpu.make_async_copy(
            h_hbm.at[sidx.at[j0 + b]], rows_v.at[b], gsem.at[b]).wait()
        scat.append(pltpu.async_copy(
            rows_v.at[b], acc.at[didx.at[j0 + b]], ssem.at[b], add=True))
    for d in scat:
        d.wait()
    for j in range(npipe * nb, CHUNKS_PER_TILE):
        pltpu.sync_copy(h_hbm.at[sidx.at[j]], rows_v.at[0])
        pltpu.sync_copy(rows_v.at[0], acc.at[didx.at[j]], add=True)

    plsc.subcore_barrier()
    pltpu.sync_copy(acc.at[rows], out_hbm.at[c, rows])


def _make_gs(width, nb):
    return pl.kernel(
        functools.partial(_gs_body, nb),
        out_type=jax.ShapeDtypeStruct((NC, N_PAD, width), jnp.float32),
        mesh=_mesh,
        compiler_params=pltpu.CompilerParams(use_tc_tiling_on_sc=False),
        scratch_types=[
            pltpu.VMEM((CHUNKS_PER_TILE, CH), jnp.int32),
            pltpu.VMEM((CHUNKS_PER_TILE, CH), jnp.int32),
            pltpu.VMEM((nb, CH, width), jnp.float32),
            pltpu.SemaphoreType.DMA((nb,)),
            pltpu.SemaphoreType.DMA((nb,)),
            pltpu.VMEM_SHARED((N_PAD, width), jnp.float32),
        ],
    )


_gs64 = _make_gs(N_CLASSES, 5)


def _inv_sqrt_deg(degs_ref, kind):
    d = degs_ref[0, kind][:, :1] + degs_ref[1, kind][:, :1]
    return lax.rsqrt(jnp.maximum(d, 1.0))


def _prescale_body(x_ref, degs_ref, ha_ref, hb_ref):
    h = x_ref[...] * _inv_sqrt_deg(degs_ref, 0)
    ha_ref[...] = h[:, :N_CLASSES]
    hb_ref[...] = h[:, N_CLASSES:]


def _mid_body(pa_ref, pb_ref, degs_ref, w1a_ref, w1b_ref, b1_ref, w2_ref,
              g_ref):
    nd = _inv_sqrt_deg(degs_ref, 1)
    t = jnp.dot((pa_ref[0] + pa_ref[1]) * nd, w1a_ref[...],
                preferred_element_type=jnp.float32)
    t += jnp.dot((pb_ref[0] + pb_ref[1]) * nd, w1b_ref[...],
                 preferred_element_type=jnp.float32)
    t = jnp.maximum(t + b1_ref[...], 0.0) * _inv_sqrt_deg(degs_ref, 0)
    g_ref[...] = jnp.dot(t, w2_ref[...], preferred_element_type=jnp.float32)


def _final_body(q_ref, degs_ref, b2_ref, o_ref):
    o_ref[...] = (q_ref[0] + q_ref[1]) * _inv_sqrt_deg(degs_ref, 1) + b2_ref[...]


_DEG_SPEC = pl.BlockSpec((NC, 2, BR, DEG_W), lambda i: (0, 0, i, 0))


def _prescale(x, degs):
    half = jax.ShapeDtypeStruct((N_NODES, N_CLASSES), jnp.float32)
    return pl.pallas_call(
        _prescale_body,
        out_shape=[half, half],
        grid=(GRID,),
        in_specs=[pl.BlockSpec((BR, D_IN), lambda i: (i, 0)), _DEG_SPEC],
        out_specs=[pl.BlockSpec((BR, N_CLASSES), lambda i: (i, 0)),
                   pl.BlockSpec((BR, N_CLASSES), lambda i: (i, 0))],
    )(x, degs)


def _mid(pa, pb, degs, W1a, W1b, b1, W2):
    return pl.pallas_call(
        _mid_body,
        out_shape=jax.ShapeDtypeStruct((N_NODES, N_CLASSES), jnp.float32),
        grid=(GRID,),
        in_specs=[
            pl.BlockSpec((NC, BR, N_CLASSES), lambda i: (0, i, 0)),
            pl.BlockSpec((NC, BR, N_CLASSES), lambda i: (0, i, 0)),
            _DEG_SPEC,
            pl.BlockSpec((N_CLASSES, D_HID), lambda i: (0, 0)),
            pl.BlockSpec((N_CLASSES, D_HID), lambda i: (0, 0)),
            pl.BlockSpec((1, D_HID), lambda i: (0, 0)),
            pl.BlockSpec((D_HID, N_CLASSES), lambda i: (0, 0)),
        ],
        out_specs=pl.BlockSpec((BR, N_CLASSES), lambda i: (i, 0)),
    )(pa, pb, degs, W1a, W1b, b1, W2)


def _final(q, degs, b2):
    return pl.pallas_call(
        _final_body,
        out_shape=jax.ShapeDtypeStruct((N_NODES, N_CLASSES), jnp.float32),
        grid=(GRID,),
        in_specs=[
            pl.BlockSpec((NC, BR, N_CLASSES), lambda i: (0, i, 0)),
            _DEG_SPEC,
            pl.BlockSpec((1, N_CLASSES), lambda i: (0, 0)),
        ],
        out_specs=pl.BlockSpec((BR, N_CLASSES), lambda i: (i, 0)),
    )(q, degs, b2)


def kernel(x, edge_index, W1, b1, W2, b2):
    ei = edge_index.astype(jnp.int32)
    src2d = ei[0].reshape(NC * NS, CHUNKS_PER_TILE, CH)
    dst2d = ei[1].reshape(NC * NS, CHUNKS_PER_TILE, CH)
    z16 = jnp.zeros((N_PAD, DEG_W), jnp.float32)
    z64 = jnp.zeros((N_PAD, N_CLASSES), jnp.float32)
    ones16 = jnp.ones((CH, DEG_W), jnp.float32)

    degs = _deg_kernel(src2d, dst2d, z16, ones16)
    h1a, h1b = _prescale(x, degs)
    p1a = _gs64(h1a, src2d, dst2d, z64)
    p1b = _gs64(h1b, src2d, dst2d, z64)
    g = _mid(p1a, p1b, degs, W1[:N_CLASSES], W1[N_CLASSES:],
             b1.reshape(1, D_HID), W2)
    p2 = _gs64(g, src2d, dst2d, z64)
    return _final(p2, degs, b2.reshape(1, N_CLASSES))
